# baseline (speedup 1.0000x reference)
import jax
import jax.numpy as jnp
from jax import lax
from jax.experimental import pallas as pl
from jax.experimental.pallas import tpu as pltpu
from jax.experimental.pallas import tpu_sc as plsc

B, N, M, C = 8, 4096, 1024, 64
TN = 1024
TC_ = 1024
TD = 512
BIG = 3.4e38


def _knn_body(t_ref, s_ref, idx_ref, w_ref):
    T = t_ref[0]
    St = jnp.transpose(s_ref[0])
    dot = lax.dot_general(T, St, (((1,), (0,)), ((), ())),
                          preferred_element_type=jnp.float32)
    t2 = jnp.sum(T * T, axis=1, keepdims=True)
    s2 = jnp.sum(St * St, axis=0, keepdims=True)
    d0 = (t2 + s2) - 2.0 * dot
    r_i = lax.broadcasted_iota(jnp.int32, (M, 3), 0)
    c_i = lax.broadcasted_iota(jnp.int32, (M, 3), 1)
    ext = jnp.where(c_i == 0, (r_i // 32).astype(jnp.float32),
                    jnp.where(c_i == 1, (r_i % 32).astype(jnp.float32), 1.0))
    dn = (((1,), (0,)), ((), ()))

    d = d0
    mvs, sts = [], []
    for k in range(3):
        mv = jnp.min(d, axis=1, keepdims=True)
        eq = d == mv
        mvs.append(mv)
        sts.append(lax.dot_general(jnp.where(eq, 1.0, 0.0), ext, dn,
                                   preferred_element_type=jnp.float32))
        if k < 2:
            d = jnp.where(eq, BIG, d)
    anytie = (jnp.any(sts[0][:, 2:3] > 1.5) | jnp.any(sts[1][:, 2:3] > 1.5)
              | jnp.any(sts[2][:, 2:3] > 1.5))

    def _fast():
        ivf = [(32.0 * s[:, 0:1] + s[:, 1:2]).astype(jnp.int32) for s in sts]
        return tuple(ivf) + tuple(mvs)

    def _slow():
        iota = lax.broadcasted_iota(jnp.int32, (TN, M), 1)
        dd = d0
        res = []
        for k in range(3):
            mv = jnp.min(dd, axis=1, keepdims=True)
            cand = jnp.where(dd == mv, iota, M)
            iv = jnp.min(cand, axis=1, keepdims=True)
            res.append((iv, mv))
            if k < 2:
                dd = jnp.where(cand == iv, BIG, dd)
        return (res[0][0], res[1][0], res[2][0],
                res[0][1], res[1][1], res[2][1])

    i1, i2, i3, m1, m2, m3 = lax.cond(anytie, _slow, _fast)
    ivs = [i1, i2, i3]
    rvs = [1.0 / (jnp.sqrt(jnp.maximum(m, 1e-12)) + 1e-8)
           for m in (m1, m2, m3)]
    rsum = rvs[0] + rvs[1] + rvs[2]
    idx_ref[0] = jnp.transpose(jnp.concatenate(ivs, axis=1))
    w_ref[0] = jnp.transpose(
        jnp.concatenate([r / rsum for r in rvs], axis=1))


def _mlpnorm_body(it0_ref, it1_ref, tf_ref, w0_ref, gb_ref, o_ref, y_scr,
                  st_scr, ss_scr):
    ph = pl.program_id(0)
    b = pl.program_id(1)
    j = pl.program_id(2)
    nt = N // TC_
    step = b * nt + j
    col = step * TC_

    @pl.when(ph == 0)
    def _():
        it = jnp.where(b < _HB, it0_ref[0], it1_ref[0])
        tf = tf_ref[0]
        W0 = w0_ref[...]
        dn = (((1,), (0,)), ((), ()))
        y = (lax.dot_general(W0[:, :C], it, dn,
                             preferred_element_type=jnp.float32)
             + lax.dot_general(W0[:, C:], tf, dn,
                               preferred_element_type=jnp.float32))
        y_scr[:, pl.ds(col, TC_)] = y
        s1 = jnp.sum(y, axis=1, keepdims=True)
        s2 = jnp.sum(y * y, axis=1, keepdims=True)
        s = jnp.concatenate([s1, s2], axis=1)

        @pl.when(step == 0)
        def _():
            st_scr[...] = s

        @pl.when(step != 0)
        def _():
            st_scr[...] += s

        @pl.when(step == B * nt - 1)
        def _():
            st = st_scr[...]
            inv_n = 1.0 / (B * N)
            mean = st[:, 0:1] * inv_n
            var = st[:, 1:2] * inv_n - mean * mean
            sc = gb_ref[:, 0:1] * lax.rsqrt(var + 1e-5)
            sh = gb_ref[:, 1:2] - mean * sc
            ss_scr[...] = jnp.concatenate([sc, sh], axis=1)

    @pl.when(ph == 1)
    def _():
        y = y_scr[:, pl.ds(col, TC_)]
        sc = ss_scr[:, 0:1]
        sh = ss_scr[:, 1:2]
        o_ref[0] = jnp.maximum(y * sc + sh, 0.0)


_NW = 32
_HB = B // 2
_WPB = _NW // _HB
_NPW = N // _WPB
_CHUNK = 512


def _interp_sc_body(sf_hbm, idx_hbm, w_hbm, out_hbm, table_v, idx_v, w_v,
                    out_v, sem):
    wid = lax.axis_index("s") * 2 + lax.axis_index("c")
    b = wid // _WPB
    base = (wid % _WPB) * _NPW
    cp = pltpu.async_copy(sf_hbm.at[b], table_v, sem)
    pltpu.sync_copy(idx_hbm.at[b, :, pl.ds(base, _NPW)], idx_v)
    pltpu.sync_copy(w_hbm.at[b, :, pl.ds(base, _NPW)], w_v)
    cp.wait()
    for chunk in range(_NPW // _CHUNK):
        coff = chunk * _CHUNK

        @plsc.parallel_loop(coff, coff + _CHUNK, 16)
        def _(s):
            i0 = idx_v[0, pl.ds(s, 16)]
            i1 = idx_v[1, pl.ds(s, 16)]
            i2 = idx_v[2, pl.ds(s, 16)]
            w0 = w_v[0, pl.ds(s, 16)]
            w1 = w_v[1, pl.ds(s, 16)]
            w2 = w_v[2, pl.ds(s, 16)]
            o = s - coff
            for c in range(C):
                off = c * M
                v = (w0 * plsc.load_gather(table_v, [i0 + off])
                     + w1 * plsc.load_gather(table_v, [i1 + off])
                     + w2 * plsc.load_gather(table_v, [i2 + off]))
                out_v[c, pl.ds(o, 16)] = v

        pltpu.sync_copy(out_v, out_hbm.at[b, :, pl.ds(base + coff, _CHUNK)])


def _interp_sc(sf_flat, idx_t, w_t):
    mesh = plsc.VectorSubcoreMesh(core_axis_name="c", subcore_axis_name="s")
    return pl.kernel(
        _interp_sc_body,
        out_type=jax.ShapeDtypeStruct((_HB, C, N), jnp.float32),
        mesh=mesh,
        scratch_types=[
            pltpu.VMEM((C * M,), jnp.float32),
            pltpu.VMEM((3, _NPW), jnp.int32),
            pltpu.VMEM((3, _NPW), jnp.float32),
            pltpu.VMEM((C, _CHUNK), jnp.float32),
            pltpu.SemaphoreType.DMA,
        ],
        compiler_params=pltpu.CompilerParams(use_tc_tiling_on_sc=False,
                                             needs_layout_passes=False),
    )(sf_flat, idx_t, w_t)


def _knn(target_h, source_h):
    return pl.pallas_call(
        _knn_body,
        grid=(_HB, N // TN),
        in_specs=[
            pl.BlockSpec((1, TN, 3), lambda b, j: (b, j, 0)),
            pl.BlockSpec((1, M, 3), lambda b, j: (b, 0, 0)),
        ],
        out_specs=[
            pl.BlockSpec((1, 3, TN), lambda b, j: (b, 0, j)),
            pl.BlockSpec((1, 3, TN), lambda b, j: (b, 0, j)),
        ],
        out_shape=[
            jax.ShapeDtypeStruct((_HB, 3, N), jnp.int32),
            jax.ShapeDtypeStruct((_HB, 3, N), jnp.float32),
        ],
    )(target_h, source_h)


def kernel(target, source, target_feats, source_feats, W0, gamma0, beta0):
    sf_flat = source_feats.reshape(B, C * M)
    idx0, w0 = _knn(target[:_HB], source[:_HB])
    interp0 = _interp_sc(sf_flat[:_HB], idx0, w0)
    idx1, w1 = _knn(target[_HB:], source[_HB:])
    interp1 = _interp_sc(sf_flat[_HB:], idx1, w1)

    gb = jnp.stack([gamma0, beta0], axis=1)

    out = pl.pallas_call(
        _mlpnorm_body,
        grid=(2, B, N // TC_),
        in_specs=[
            pl.BlockSpec((1, C, TC_),
                         lambda p, b, j: (b % _HB, 0, j * (1 - p))),
            pl.BlockSpec((1, C, TC_),
                         lambda p, b, j: (b % _HB, 0, j * (1 - p))),
            pl.BlockSpec((1, C, TC_),
                         lambda p, b, j: (b * (1 - p), 0, j * (1 - p))),
            pl.BlockSpec((C, 2 * C), lambda p, b, j: (0, 0)),
            pl.BlockSpec((C, 2), lambda p, b, j: (0, 0)),
        ],
        out_specs=pl.BlockSpec((1, C, TC_), lambda p, b, j: (b, 0, j)),
        out_shape=jax.ShapeDtypeStruct((B, C, N), jnp.float32),
        scratch_shapes=[
            pltpu.VMEM((C, B * N), jnp.float32),
            pltpu.VMEM((C, 2), jnp.float32),
            pltpu.VMEM((C, 2), jnp.float32),
        ],
    )(interp0, interp1, target_feats, W0, gb)
    return out


# speedup vs baseline: 1.1614x; 1.1614x over previous
import jax
import jax.numpy as jnp
from jax import lax
from jax.experimental import pallas as pl
from jax.experimental.pallas import tpu as pltpu
from jax.experimental.pallas import tpu_sc as plsc

B, N, M, C = 8, 4096, 1024, 64
TN = 1024
TC_ = 1024
TD = 512
BIG = 3.4e38


def _knn_body(t_ref, s_ref, idx_ref, w_ref):
    T = t_ref[0]
    St = jnp.transpose(s_ref[0])
    dot = lax.dot_general(T, St, (((1,), (0,)), ((), ())),
                          preferred_element_type=jnp.float32)
    t2 = jnp.sum(T * T, axis=1, keepdims=True)
    s2 = jnp.sum(St * St, axis=0, keepdims=True)
    d = (t2 + s2) - 2.0 * dot
    iota = lax.broadcasted_iota(jnp.int32, (TN, M), 1)
    ivs, rvs = [], []
    for k in range(3):
        mv = jnp.min(d, axis=1, keepdims=True)
        cand = jnp.where(d == mv, iota, M)
        iv = jnp.min(cand, axis=1, keepdims=True)
        if k < 2:
            d = jnp.where(cand == iv, BIG, d)
        dist = jnp.sqrt(jnp.maximum(mv, 1e-12))
        ivs.append(iv)
        rvs.append(1.0 / (dist + 1e-8))
    rsum = rvs[0] + rvs[1] + rvs[2]
    idx_ref[0] = jnp.transpose(jnp.concatenate(ivs, axis=1))
    w_ref[0] = jnp.transpose(
        jnp.concatenate([r / rsum for r in rvs], axis=1))


def _mlpnorm_body(it0_ref, it1_ref, tf_ref, w0_ref, gb_ref, o_ref, y_scr,
                  st_scr, ss_scr):
    ph = pl.program_id(0)
    b = pl.program_id(1)
    j = pl.program_id(2)
    nt = N // TC_
    step = b * nt + j
    col = step * TC_

    @pl.when(ph == 0)
    def _():
        it = jnp.where(b < _HB, it0_ref[0], it1_ref[0])
        tf = tf_ref[0]
        W0 = w0_ref[...]
        dn = (((1,), (0,)), ((), ()))
        y = (lax.dot_general(W0[:, :C], it, dn,
                             preferred_element_type=jnp.float32)
             + lax.dot_general(W0[:, C:], tf, dn,
                               preferred_element_type=jnp.float32))
        y_scr[:, pl.ds(col, TC_)] = y
        s1 = jnp.sum(y, axis=1, keepdims=True)
        s2 = jnp.sum(y * y, axis=1, keepdims=True)
        s = jnp.concatenate([s1, s2], axis=1)

        @pl.when(step == 0)
        def _():
            st_scr[...] = s

        @pl.when(step != 0)
        def _():
            st_scr[...] += s

        @pl.when(step == B * nt - 1)
        def _():
            st = st_scr[...]
            inv_n = 1.0 / (B * N)
            mean = st[:, 0:1] * inv_n
            var = st[:, 1:2] * inv_n - mean * mean
            sc = gb_ref[:, 0:1] * lax.rsqrt(var + 1e-5)
            sh = gb_ref[:, 1:2] - mean * sc
            ss_scr[...] = jnp.concatenate([sc, sh], axis=1)

    @pl.when(ph == 1)
    def _():
        y = y_scr[:, pl.ds(col, TC_)]
        sc = ss_scr[:, 0:1]
        sh = ss_scr[:, 1:2]
        o_ref[0] = jnp.maximum(y * sc + sh, 0.0)


_NW = 32
_HB = B // 2
_WPB = _NW // _HB
_NPW = N // _WPB
_CHUNK = 512


def _interp_sc_body(sf_hbm, idx_hbm, w_hbm, out_hbm, table_v, idx_v, w_v,
                    out_v, sem):
    wid = lax.axis_index("s") * 2 + lax.axis_index("c")
    b = wid // _WPB
    base = (wid % _WPB) * _NPW
    cp = pltpu.async_copy(sf_hbm.at[b], table_v, sem)
    pltpu.sync_copy(idx_hbm.at[b, :, pl.ds(base, _NPW)], idx_v)
    pltpu.sync_copy(w_hbm.at[b, :, pl.ds(base, _NPW)], w_v)
    cp.wait()
    for chunk in range(_NPW // _CHUNK):
        coff = chunk * _CHUNK

        @plsc.parallel_loop(coff, coff + _CHUNK, 16)
        def _(s):
            i0 = idx_v[0, pl.ds(s, 16)]
            i1 = idx_v[1, pl.ds(s, 16)]
            i2 = idx_v[2, pl.ds(s, 16)]
            w0 = w_v[0, pl.ds(s, 16)]
            w1 = w_v[1, pl.ds(s, 16)]
            w2 = w_v[2, pl.ds(s, 16)]
            o = s - coff
            for c in range(C):
                off = c * M
                v = (w0 * plsc.load_gather(table_v, [i0 + off])
                     + w1 * plsc.load_gather(table_v, [i1 + off])
                     + w2 * plsc.load_gather(table_v, [i2 + off]))
                out_v[c, pl.ds(o, 16)] = v

        pltpu.sync_copy(out_v, out_hbm.at[b, :, pl.ds(base + coff, _CHUNK)])


def _interp_sc(sf_flat, idx_t, w_t):
    mesh = plsc.VectorSubcoreMesh(core_axis_name="c", subcore_axis_name="s")
    return pl.kernel(
        _interp_sc_body,
        out_type=jax.ShapeDtypeStruct((_HB, C, N), jnp.float32),
        mesh=mesh,
        scratch_types=[
            pltpu.VMEM((C * M,), jnp.float32),
            pltpu.VMEM((3, _NPW), jnp.int32),
            pltpu.VMEM((3, _NPW), jnp.float32),
            pltpu.VMEM((C, _CHUNK), jnp.float32),
            pltpu.SemaphoreType.DMA,
        ],
        compiler_params=pltpu.CompilerParams(use_tc_tiling_on_sc=False,
                                             needs_layout_passes=False),
    )(sf_flat, idx_t, w_t)


def _knn(target_h, source_h):
    return pl.pallas_call(
        _knn_body,
        grid=(_HB, N // TN),
        in_specs=[
            pl.BlockSpec((1, TN, 3), lambda b, j: (b, j, 0)),
            pl.BlockSpec((1, M, 3), lambda b, j: (b, 0, 0)),
        ],
        out_specs=[
            pl.BlockSpec((1, 3, TN), lambda b, j: (b, 0, j)),
            pl.BlockSpec((1, 3, TN), lambda b, j: (b, 0, j)),
        ],
        out_shape=[
            jax.ShapeDtypeStruct((_HB, 3, N), jnp.int32),
            jax.ShapeDtypeStruct((_HB, 3, N), jnp.float32),
        ],
    )(target_h, source_h)


def kernel(target, source, target_feats, source_feats, W0, gamma0, beta0):
    sf_flat = source_feats.reshape(B, C * M)
    idx0, w0 = _knn(target[:_HB], source[:_HB])
    interp0 = _interp_sc(sf_flat[:_HB], idx0, w0)
    idx1, w1 = _knn(target[_HB:], source[_HB:])
    interp1 = _interp_sc(sf_flat[_HB:], idx1, w1)

    gb = jnp.stack([gamma0, beta0], axis=1)

    out = pl.pallas_call(
        _mlpnorm_body,
        grid=(2, B, N // TC_),
        in_specs=[
            pl.BlockSpec((1, C, TC_),
                         lambda p, b, j: (b % _HB, 0, j * (1 - p))),
            pl.BlockSpec((1, C, TC_),
                         lambda p, b, j: (b % _HB, 0, j * (1 - p))),
            pl.BlockSpec((1, C, TC_),
                         lambda p, b, j: (b * (1 - p), 0, j * (1 - p))),
            pl.BlockSpec((C, 2 * C), lambda p, b, j: (0, 0)),
            pl.BlockSpec((C, 2), lambda p, b, j: (0, 0)),
        ],
        out_specs=pl.BlockSpec((1, C, TC_), lambda p, b, j: (b, 0, j)),
        out_shape=jax.ShapeDtypeStruct((B, C, N), jnp.float32),
        scratch_shapes=[
            pltpu.VMEM((C, B * N), jnp.float32),
            pltpu.VMEM((C, 2), jnp.float32),
            pltpu.VMEM((C, 2), jnp.float32),
        ],
    )(interp0, interp1, target_feats, W0, gb)
    return out


# speedup vs baseline: 1.1620x; 1.0005x over previous
import jax
import jax.numpy as jnp
from jax import lax
from jax.experimental import pallas as pl
from jax.experimental.pallas import tpu as pltpu
from jax.experimental.pallas import tpu_sc as plsc

B, N, M, C = 8, 4096, 1024, 64
TN = 1024
TC_ = 1024
BIG = 3.4e38


def _knn_body(t_ref, s_ref, idx_ref, w_ref):
    T = t_ref[0]
    St = jnp.transpose(s_ref[0])
    dot = lax.dot_general(T, St, (((1,), (0,)), ((), ())),
                          preferred_element_type=jnp.float32)
    t2 = jnp.sum(T * T, axis=1, keepdims=True)
    s2 = jnp.sum(St * St, axis=0, keepdims=True)
    d = (t2 + s2) - 2.0 * dot
    iota = lax.broadcasted_iota(jnp.int32, (TN, M), 1)
    ivs, rvs = [], []
    for k in range(3):
        mv = jnp.min(d, axis=1, keepdims=True)
        cand = jnp.where(d == mv, iota, M)
        iv = jnp.min(cand, axis=1, keepdims=True)
        if k < 2:
            d = jnp.where(cand == iv, BIG, d)
        dist = jnp.sqrt(jnp.maximum(mv, 1e-12))
        ivs.append(iv)
        rvs.append(1.0 / (dist + 1e-8))
    rsum = rvs[0] + rvs[1] + rvs[2]
    idx_ref[0] = jnp.transpose(jnp.concatenate(ivs, axis=1))
    w_ref[0] = jnp.transpose(
        jnp.concatenate([r / rsum for r in rvs], axis=1))


def _mlpnorm_body(it0_ref, it1_ref, tf_ref, w0_ref, gb_ref, o_ref, y_scr,
                  st_scr, ss_scr):
    ph = pl.program_id(0)
    b = pl.program_id(1)
    j = pl.program_id(2)
    nt = N // TC_
    step = b * nt + j
    col = step * TC_

    @pl.when(ph == 0)
    def _():
        it = jnp.where(b < _HB, it0_ref[0], it1_ref[0])
        tf = tf_ref[0]
        W0 = w0_ref[...]
        dn = (((1,), (0,)), ((), ()))
        y = (lax.dot_general(W0[:, :C], it, dn,
                             preferred_element_type=jnp.float32)
             + lax.dot_general(W0[:, C:], tf, dn,
                               preferred_element_type=jnp.float32))
        y_scr[:, pl.ds(col, TC_)] = y
        s1 = jnp.sum(y, axis=1, keepdims=True)
        s2 = jnp.sum(y * y, axis=1, keepdims=True)
        s = jnp.concatenate([s1, s2], axis=1)

        @pl.when(step == 0)
        def _():
            st_scr[...] = s

        @pl.when(step != 0)
        def _():
            st_scr[...] += s

        @pl.when(step == B * nt - 1)
        def _():
            st = st_scr[...]
            inv_n = 1.0 / (B * N)
            mean = st[:, 0:1] * inv_n
            var = st[:, 1:2] * inv_n - mean * mean
            sc = gb_ref[:, 0:1] * lax.rsqrt(var + 1e-5)
            sh = gb_ref[:, 1:2] - mean * sc
            ss_scr[...] = jnp.concatenate([sc, sh], axis=1)

    @pl.when(ph == 1)
    def _():
        y = y_scr[:, pl.ds(col, TC_)]
        sc = ss_scr[:, 0:1]
        sh = ss_scr[:, 1:2]
        o_ref[0] = jnp.maximum(y * sc + sh, 0.0)


_NW = 32
_HB = B // 2
_WPB = _NW // _HB
_NPW = N // _WPB
_CHUNK = 512


def _interp_sc_body(sf_hbm, idx_hbm, w_hbm, out_hbm, table_v, idx_v, w_v,
                    out_v, sem):
    wid = lax.axis_index("s") * 2 + lax.axis_index("c")
    b = wid // _WPB
    base = (wid % _WPB) * _NPW
    cp = pltpu.async_copy(sf_hbm.at[b], table_v, sem)
    pltpu.sync_copy(idx_hbm.at[b, :, pl.ds(base, _NPW)], idx_v)
    pltpu.sync_copy(w_hbm.at[b, :, pl.ds(base, _NPW)], w_v)
    cp.wait()
    for chunk in range(_NPW // _CHUNK):
        coff = chunk * _CHUNK

        @plsc.parallel_loop(coff, coff + _CHUNK, 16)
        def _(s):
            i0 = idx_v[0, pl.ds(s, 16)]
            i1 = idx_v[1, pl.ds(s, 16)]
            i2 = idx_v[2, pl.ds(s, 16)]
            w0 = w_v[0, pl.ds(s, 16)]
            w1 = w_v[1, pl.ds(s, 16)]
            w2 = w_v[2, pl.ds(s, 16)]
            o = s - coff
            for c in range(C):
                off = c * M
                v = (w0 * plsc.load_gather(table_v, [i0 + off])
                     + w1 * plsc.load_gather(table_v, [i1 + off])
                     + w2 * plsc.load_gather(table_v, [i2 + off]))
                out_v[c, pl.ds(o, 16)] = v

        pltpu.sync_copy(out_v, out_hbm.at[b, :, pl.ds(base + coff, _CHUNK)])


def _interp_sc(sf_flat, idx_t, w_t):
    mesh = plsc.VectorSubcoreMesh(core_axis_name="c", subcore_axis_name="s")
    return pl.kernel(
        _interp_sc_body,
        out_type=jax.ShapeDtypeStruct((_HB, C, N), jnp.float32),
        mesh=mesh,
        scratch_types=[
            pltpu.VMEM((C * M,), jnp.float32),
            pltpu.VMEM((3, _NPW), jnp.int32),
            pltpu.VMEM((3, _NPW), jnp.float32),
            pltpu.VMEM((C, _CHUNK), jnp.float32),
            pltpu.SemaphoreType.DMA,
        ],
        compiler_params=pltpu.CompilerParams(use_tc_tiling_on_sc=False,
                                             needs_layout_passes=False),
    )(sf_flat, idx_t, w_t)


def _knn(target_h, source_h):
    return pl.pallas_call(
        _knn_body,
        grid=(_HB, N // TN),
        in_specs=[
            pl.BlockSpec((1, TN, 3), lambda b, j: (b, j, 0)),
            pl.BlockSpec((1, M, 3), lambda b, j: (b, 0, 0)),
        ],
        out_specs=[
            pl.BlockSpec((1, 3, TN), lambda b, j: (b, 0, j)),
            pl.BlockSpec((1, 3, TN), lambda b, j: (b, 0, j)),
        ],
        out_shape=[
            jax.ShapeDtypeStruct((_HB, 3, N), jnp.int32),
            jax.ShapeDtypeStruct((_HB, 3, N), jnp.float32),
        ],
    )(target_h, source_h)


def kernel(target, source, target_feats, source_feats, W0, gamma0, beta0):
    sf_flat = source_feats.reshape(B, C * M)
    idx0, w0 = _knn(target[:_HB], source[:_HB])
    interp0 = _interp_sc(sf_flat[:_HB], idx0, w0)
    idx1, w1 = _knn(target[_HB:], source[_HB:])
    interp1 = _interp_sc(sf_flat[_HB:], idx1, w1)

    gb = jnp.stack([gamma0, beta0], axis=1)

    out = pl.pallas_call(
        _mlpnorm_body,
        grid=(2, B, N // TC_),
        in_specs=[
            pl.BlockSpec((1, C, TC_),
                         lambda p, b, j: (b % _HB, 0, j * (1 - p))),
            pl.BlockSpec((1, C, TC_),
                         lambda p, b, j: (b % _HB, 0, j * (1 - p))),
            pl.BlockSpec((1, C, TC_),
                         lambda p, b, j: (b * (1 - p), 0, j * (1 - p))),
            pl.BlockSpec((C, 2 * C), lambda p, b, j: (0, 0)),
            pl.BlockSpec((C, 2), lambda p, b, j: (0, 0)),
        ],
        out_specs=pl.BlockSpec((1, C, TC_), lambda p, b, j: (b, 0, j)),
        out_shape=jax.ShapeDtypeStruct((B, C, N), jnp.float32),
        scratch_shapes=[
            pltpu.VMEM((C, B * N), jnp.float32),
            pltpu.VMEM((C, 2), jnp.float32),
            pltpu.VMEM((C, 2), jnp.float32),
        ],
    )(interp0, interp1, target_feats, W0, gb)
    return out
